# Initial kernel scaffold; baseline (speedup 1.0000x reference)
import jax
import jax.numpy as jnp
from jax.experimental import pallas as pl


def kernel(h, Wq, bq, Wk, bk):
    raise NotImplementedError("write your pallas kernel here")


# speedup vs baseline: 2.9340x; 2.9340x over previous
import functools
import math

import jax
import jax.numpy as jnp
from jax.experimental import pallas as pl
from jax.experimental.pallas import tpu as pltpu

_TOPK = 32


def _body(h_ref, w_ref, b_ref, idx_ref, val_ref, scores_ref):
    h = h_ref[0]
    C = h.shape[0]
    D = h.shape[1]
    qk = jnp.dot(h, w_ref[...], preferred_element_type=jnp.float32) + b_ref[...]
    q = qk[:, :D]
    k = qk[:, D:]
    s = jax.lax.dot_general(q, k, (((1,), (1,)), ((), ())),
                            preferred_element_type=jnp.float32)
    s = s * (1.0 / math.sqrt(D))
    row = jax.lax.broadcasted_iota(jnp.int32, (C, C), 0)
    col = jax.lax.broadcasted_iota(jnp.int32, (C, C), 1)
    neg_inf = jnp.float32(-jnp.inf)
    s = jnp.where(row == col, neg_inf, s)
    scores_ref[0] = s

    cur = s
    vals = []
    idxs = []
    for _ in range(_TOPK):
        m = jnp.max(cur, axis=1, keepdims=True)
        hit = cur == m
        idx = jnp.min(jnp.where(hit, col, C), axis=1, keepdims=True)
        vals.append(m)
        idxs.append(idx)
        cur = jnp.where(col == idx, neg_inf, cur)
    val_ref[0] = jnp.concatenate(vals, axis=1)
    idx_ref[0] = jnp.concatenate(idxs, axis=1)


@jax.jit
def kernel(h, Wq, bq, Wk, bk):
    B, C, D = h.shape
    w = jnp.concatenate([Wq, Wk], axis=1)
    b = jnp.concatenate([bq, bk], axis=0)[None, :]
    kcall = pl.pallas_call(
        _body,
        grid=(B,),
        in_specs=[
            pl.BlockSpec((1, C, D), lambda i: (i, 0, 0)),
            pl.BlockSpec((D, 2 * D), lambda i: (0, 0)),
            pl.BlockSpec((1, 2 * D), lambda i: (0, 0)),
        ],
        out_specs=[
            pl.BlockSpec((1, C, _TOPK), lambda i: (i, 0, 0)),
            pl.BlockSpec((1, C, _TOPK), lambda i: (i, 0, 0)),
            pl.BlockSpec((1, C, C), lambda i: (i, 0, 0)),
        ],
        out_shape=[
            jax.ShapeDtypeStruct((B, C, _TOPK), jnp.int32),
            jax.ShapeDtypeStruct((B, C, _TOPK), jnp.float32),
            jax.ShapeDtypeStruct((B, C, C), jnp.float32),
        ],
    )
    idx, val, scores = kcall(h, w, b)
    return (idx, val, scores)


# speedup vs baseline: 4.2348x; 1.4434x over previous
import functools
import math

import jax
import jax.numpy as jnp
from jax.experimental import pallas as pl
from jax.experimental.pallas import tpu as pltpu

_TOPK = 32


def _body(h_ref, w_ref, b_ref, idx_ref, val_ref, scores_ref):
    h = h_ref[0]
    C = h.shape[0]
    D = h.shape[1]
    qk = jnp.dot(h, w_ref[...], preferred_element_type=jnp.float32) + b_ref[...]
    q = qk[:, :D]
    k = qk[:, D:]
    s = jax.lax.dot_general(q, k, (((1,), (1,)), ((), ())),
                            preferred_element_type=jnp.float32)
    s = s * (1.0 / math.sqrt(D))
    row = jax.lax.broadcasted_iota(jnp.int32, (C, C), 0)
    col = jax.lax.broadcasted_iota(jnp.int32, (C, C), 1)
    colf = col.astype(jnp.float32)
    neg_inf = jnp.float32(-jnp.inf)
    big = jnp.float32(C)
    s = jnp.where(row == col, neg_inf, s)
    scores_ref[0] = s

    cur = s
    m = jnp.max(cur, axis=1, keepdims=True)
    vals = []
    idxfs = []
    for t in range(_TOPK):
        idxf = jnp.min(jnp.where(cur == m, colf, big), axis=1, keepdims=True)
        vals.append(m)
        idxfs.append(idxf)
        if t < _TOPK - 1:
            cur = jnp.where(colf == idxf, neg_inf, cur)
            m = jnp.max(cur, axis=1, keepdims=True)
    val_ref[0] = jnp.concatenate(vals, axis=1)
    idx_ref[0] = jnp.concatenate(idxfs, axis=1).astype(jnp.int32)


@jax.jit
def kernel(h, Wq, bq, Wk, bk):
    B, C, D = h.shape
    w = jnp.concatenate([Wq, Wk], axis=1)
    b = jnp.concatenate([bq, bk], axis=0)[None, :]
    kcall = pl.pallas_call(
        _body,
        grid=(B,),
        in_specs=[
            pl.BlockSpec((1, C, D), lambda i: (i, 0, 0)),
            pl.BlockSpec((D, 2 * D), lambda i: (0, 0)),
            pl.BlockSpec((1, 2 * D), lambda i: (0, 0)),
        ],
        out_specs=[
            pl.BlockSpec((1, C, _TOPK), lambda i: (i, 0, 0)),
            pl.BlockSpec((1, C, _TOPK), lambda i: (i, 0, 0)),
            pl.BlockSpec((1, C, C), lambda i: (i, 0, 0)),
        ],
        out_shape=[
            jax.ShapeDtypeStruct((B, C, _TOPK), jnp.int32),
            jax.ShapeDtypeStruct((B, C, _TOPK), jnp.float32),
            jax.ShapeDtypeStruct((B, C, C), jnp.float32),
        ],
    )
    idx, val, scores = kcall(h, w, b)
    return (idx, val, scores)
